# baseline (speedup 1.0000x reference)
import functools

import jax
import jax.numpy as jnp
from jax import lax
from jax.experimental import pallas as pl
from jax.experimental.pallas import tpu as pltpu
from jax.experimental.pallas import tpu_sc as plsc

N = 10000
E = 320000
D = 128

NC = 2
NS = 16
NW = NC * NS
C = 128
NCHUNK = 79
EPW = C * NCHUNK
EP = EPW * NW
PADW = EPW - E // NW
NT = N + NS
RSTRIPE = 640
ZR = 40
CNT_CH = 2000


def _sc_body(x_hbm, ei_hbm, agg_out, cnt_out,
             idx0_v, idx1_v, src0_v, src1_v, dst0_v, dst1_v,
             rows0_v, rows1_v, ones_v, zbuf, zc,
             agg_sh, cnt_sh, gsem, isem, osem):
    s = lax.axis_index("s")
    c = lax.axis_index("c")
    wid = s * NC + c

    zero16 = jnp.zeros((16,), jnp.float32)
    one16 = jnp.ones((16,), jnp.float32)

    def fill_zbuf(i, _):
        for j in range(D // 16):
            zbuf[i, pl.ds(j * 16, 16)] = zero16
        return 0
    lax.fori_loop(0, ZR, fill_zbuf, 0)

    def fill_ones(k, _):
        ones_v[pl.ds(k * 16, 16)] = one16
        return 0
    lax.fori_loop(0, C // 16, fill_ones, 0)

    def fill_zc(k, _):
        zc[pl.ds(k * 16, 16)] = zero16
        return 0
    lax.fori_loop(0, CNT_CH // 16, fill_zc, 0)

    row0 = s * RSTRIPE
    nzero = jnp.where(s == NS - 1, (N - (NS - 1) * RSTRIPE) // ZR,
                      RSTRIPE // ZR)

    def zero_stripe(r, _):
        pltpu.sync_copy(zbuf, agg_sh.at[pl.ds(row0 + r * ZR, ZR), :])
        return 0
    lax.fori_loop(0, nzero, zero_stripe, 0)

    @pl.when(s == 0)
    def _():
        for i in range(N // CNT_CH):
            pltpu.sync_copy(zc, cnt_sh.at[pl.ds(i * CNT_CH, CNT_CH)])

    plsc.subcore_barrier()

    base0 = wid * EPW
    idxs = (idx0_v, idx1_v)
    srcs = (src0_v, src1_v)
    dsts = (dst0_v, dst1_v)
    rows = (rows0_v, rows1_v)
    isems = (isem.at[0], isem.at[1])
    gsems = (gsem.at[0], gsem.at[1])

    def load_idx(i, p):
        base = base0 + i * C
        pltpu.async_copy(ei_hbm.at[:, pl.ds(base, C)], idxs[p], isems[p])

    def wait_split_idx(p):
        pltpu.make_async_copy(ei_hbm.at[:, pl.ds(0, C)], idxs[p],
                              isems[p]).wait()
        for k in range(C // 16):
            srcs[p][pl.ds(k * 16, 16)] = idxs[p][0, pl.ds(k * 16, 16)]
            dsts[p][pl.ds(k * 16, 16)] = idxs[p][1, pl.ds(k * 16, 16)]

    load_idx(0, 0)
    wait_split_idx(0)
    pltpu.async_copy(x_hbm.at[srcs[0]], rows[0], gsems[0])
    load_idx(1, 1)

    def outer(k, _):
        for p in range(2):
            i = 2 * k + p

            @pl.when(i < NCHUNK)
            def _():
                pltpu.make_async_copy(x_hbm.at[srcs[p]], rows[p],
                                      gsems[p]).wait()

                @pl.when(i + 1 < NCHUNK)
                def _():
                    wait_split_idx(1 - p)
                    pltpu.async_copy(x_hbm.at[srcs[1 - p]], rows[1 - p],
                                     gsems[1 - p])

                pltpu.sync_copy(rows[p], agg_sh.at[dsts[p]], add=True)
                pltpu.sync_copy(ones_v, cnt_sh.at[dsts[p]], add=True)

                @pl.when(i + 2 < NCHUNK)
                def _():
                    load_idx(i + 2, p)
        return 0
    lax.fori_loop(0, (NCHUNK + 1) // 2, outer, 0)

    plsc.subcore_barrier()

    nout = jnp.where(s == NS - 1, (N - (NS - 1) * RSTRIPE) // ZR,
                     RSTRIPE // ZR)

    def out_stripe(r, _):
        rb = row0 + r * ZR
        pltpu.sync_copy(agg_sh.at[pl.ds(rb, ZR), :], zbuf)
        pltpu.sync_copy(zbuf, agg_out.at[c, pl.ds(rb, ZR), :])
        return 0
    lax.fori_loop(0, nout, out_stripe, 0)

    @pl.when(s == 0)
    def _():
        for i in range(N // CNT_CH):
            pltpu.sync_copy(cnt_sh.at[pl.ds(i * CNT_CH, CNT_CH)], zc)
            pltpu.sync_copy(zc, cnt_out.at[pl.ds(c * N + i * CNT_CH, CNT_CH)])


_sc_aggregate = functools.partial(
    pl.kernel,
    out_type=[
        jax.ShapeDtypeStruct((NC, N, D), jnp.float32),
        jax.ShapeDtypeStruct((NC * N,), jnp.float32),
    ],
    mesh=plsc.VectorSubcoreMesh(core_axis_name="c", subcore_axis_name="s"),
    scratch_types=[
        pltpu.VMEM((2, C), jnp.int32),
        pltpu.VMEM((2, C), jnp.int32),
        pltpu.VMEM((C,), jnp.int32),
        pltpu.VMEM((C,), jnp.int32),
        pltpu.VMEM((C,), jnp.int32),
        pltpu.VMEM((C,), jnp.int32),
        pltpu.VMEM((C, D), jnp.float32),
        pltpu.VMEM((C, D), jnp.float32),
        pltpu.VMEM((C,), jnp.float32),
        pltpu.VMEM((ZR, D), jnp.float32),
        pltpu.VMEM((CNT_CH,), jnp.float32),
        pltpu.VMEM_SHARED((NT, D), jnp.float32),
        pltpu.VMEM_SHARED((NT,), jnp.float32),
        pltpu.SemaphoreType.DMA((2,)),
        pltpu.SemaphoreType.DMA((2,)),
        pltpu.SemaphoreType.DMA((2,)),
    ],
)(_sc_body)


B = 2000


def _tc_body(agg_ref, cnt_ref, x_ref, rr_ref, wl_ref, bl_ref,
             wr_ref, ws_ref, bs_ref, alpha_ref, out_ref):
    agg = agg_ref[0] + agg_ref[1]
    cnt = cnt_ref[pl.ds(0, N)] + cnt_ref[pl.ds(N, N)]
    mean = agg / jnp.maximum(cnt, 1.0)[:, None]
    h = (jnp.dot(mean, wl_ref[...], preferred_element_type=jnp.float32)
         + bl_ref[...]
         + jnp.dot(x_ref[...], wr_ref[...], preferred_element_type=jnp.float32))
    h = jnp.maximum(h, 0.0) + x_ref[...]
    score = jnp.sum(h * ws_ref[...], axis=1) + bs_ref[0, 0]
    a = jax.nn.sigmoid(alpha_ref[0, 0])
    out_ref[...] = a * rr_ref[...] + (1.0 - a) * score


def _tc_finish(agg, cnt_flat, x, rr, W_l, bl2, W_r, ws2, bs2, alpha2):
    return pl.pallas_call(
        _tc_body,
        out_shape=jax.ShapeDtypeStruct((N,), jnp.float32),
    )(agg, cnt_flat, x, rr, W_l, bl2, W_r, ws2, bs2, alpha2)


def kernel(x, edge_index, reranker_scores, W_l, b_l, W_r, W_score, b_score,
           alpha):
    eip = jnp.pad(edge_index, ((0, 0), (0, EP - E)))
    trash = N + jnp.arange(EP - E, dtype=jnp.int32) % NS
    eip = eip.at[1, E:].set(trash)
    agg_parts, cnt_parts = _sc_aggregate(x, eip)
    return _tc_finish(
        agg_parts,
        cnt_parts,
        x,
        reranker_scores,
        W_l,
        b_l.reshape(1, D),
        W_r,
        W_score.reshape(1, D),
        b_score.reshape(1, 1),
        alpha.reshape(1, 1),
    )


# speedup vs baseline: 1.6582x; 1.6582x over previous
import functools

import jax
import jax.numpy as jnp
from jax import lax
from jax.experimental import pallas as pl
from jax.experimental.pallas import tpu as pltpu
from jax.experimental.pallas import tpu_sc as plsc

N = 10000
E = 320000
D = 128

NC = 2
NS = 16
NW = NC * NS
C = 80
NCHUNK = 125
EPW = C * NCHUNK
EP = EPW * NW
PADW = EPW - E // NW
NT = N + NS
RSTRIPE = 640
ZR = 40
CNT_CH = 2000


def _sc_body(x_hbm, src_hbm, dst_hbm, agg_out, cnt_out,
             src0_v, src1_v, dst0_v, dst1_v,
             rows0_v, rows1_v, ones_v, zbuf, zc,
             agg_sh, cnt_sh, gsem, isem, osem):
    s = lax.axis_index("s")
    c = lax.axis_index("c")
    wid = s * NC + c

    zero16 = jnp.zeros((16,), jnp.float32)
    one16 = jnp.ones((16,), jnp.float32)

    def fill_zbuf(i, _):
        for j in range(D // 16):
            zbuf[i, pl.ds(j * 16, 16)] = zero16
        return 0
    lax.fori_loop(0, ZR, fill_zbuf, 0)

    def fill_ones(k, _):
        ones_v[pl.ds(k * 16, 16)] = one16
        return 0
    lax.fori_loop(0, C // 16, fill_ones, 0)

    def fill_zc(k, _):
        zc[pl.ds(k * 16, 16)] = zero16
        return 0
    lax.fori_loop(0, CNT_CH // 16, fill_zc, 0)

    row0 = s * RSTRIPE
    nzero = jnp.where(s == NS - 1, (N - (NS - 1) * RSTRIPE) // ZR,
                      RSTRIPE // ZR)

    def zero_stripe(r, _):
        pltpu.sync_copy(zbuf, agg_sh.at[pl.ds(row0 + r * ZR, ZR), :])
        return 0
    lax.fori_loop(0, nzero, zero_stripe, 0)

    @pl.when(s == 0)
    def _():
        for i in range(N // CNT_CH):
            pltpu.sync_copy(zc, cnt_sh.at[pl.ds(i * CNT_CH, CNT_CH)])

    plsc.subcore_barrier()

    base0 = wid * EPW
    srcs = (src0_v, src1_v)
    dsts = (dst0_v, dst1_v)
    rows = (rows0_v, rows1_v)
    isems = (isem.at[0], isem.at[1])
    gsems = (gsem.at[0], gsem.at[1])

    def load_idx(i, p):
        base = base0 + i * C
        pltpu.async_copy(src_hbm.at[pl.ds(base, C)], srcs[p], isems[p])
        pltpu.async_copy(dst_hbm.at[pl.ds(base, C)], dsts[p], isems[p])

    def wait_idx(p):
        pltpu.make_async_copy(src_hbm.at[pl.ds(0, C)], srcs[p], isems[p]).wait()
        pltpu.make_async_copy(dst_hbm.at[pl.ds(0, C)], dsts[p], isems[p]).wait()

    load_idx(0, 0)
    wait_idx(0)
    pltpu.async_copy(x_hbm.at[srcs[0]], rows[0], gsems[0])
    load_idx(1, 1)

    def outer(k, _):
        for p in range(2):
            i = 2 * k + p

            @pl.when(i < NCHUNK)
            def _():
                pltpu.make_async_copy(x_hbm.at[srcs[p]], rows[p],
                                      gsems[p]).wait()

                @pl.when(i + 1 < NCHUNK)
                def _():
                    wait_idx(1 - p)
                    pltpu.async_copy(x_hbm.at[srcs[1 - p]], rows[1 - p],
                                     gsems[1 - p])

                pltpu.sync_copy(rows[p], agg_sh.at[dsts[p]], add=True)
                pltpu.sync_copy(ones_v, cnt_sh.at[dsts[p]], add=True)

                @pl.when(i + 2 < NCHUNK)
                def _():
                    load_idx(i + 2, p)
        return 0
    lax.fori_loop(0, (NCHUNK + 1) // 2, outer, 0)

    plsc.subcore_barrier()

    nout = jnp.where(s == NS - 1, (N - (NS - 1) * RSTRIPE) // ZR,
                     RSTRIPE // ZR)

    def out_stripe(r, _):
        rb = row0 + r * ZR
        pltpu.sync_copy(agg_sh.at[pl.ds(rb, ZR), :], zbuf)
        pltpu.sync_copy(zbuf, agg_out.at[c, pl.ds(rb, ZR), :])
        return 0
    lax.fori_loop(0, nout, out_stripe, 0)

    @pl.when(s == 0)
    def _():
        for i in range(N // CNT_CH):
            pltpu.sync_copy(cnt_sh.at[pl.ds(i * CNT_CH, CNT_CH)], zc)
            pltpu.sync_copy(zc, cnt_out.at[pl.ds(c * N + i * CNT_CH, CNT_CH)])


_sc_aggregate = functools.partial(
    pl.kernel,
    out_type=[
        jax.ShapeDtypeStruct((NC, N, D), jnp.float32),
        jax.ShapeDtypeStruct((NC * N,), jnp.float32),
    ],
    mesh=plsc.VectorSubcoreMesh(core_axis_name="c", subcore_axis_name="s"),
    scratch_types=[
        pltpu.VMEM((C,), jnp.int32),
        pltpu.VMEM((C,), jnp.int32),
        pltpu.VMEM((C,), jnp.int32),
        pltpu.VMEM((C,), jnp.int32),
        pltpu.VMEM((C, D), jnp.float32),
        pltpu.VMEM((C, D), jnp.float32),
        pltpu.VMEM((C,), jnp.float32),
        pltpu.VMEM((ZR, D), jnp.float32),
        pltpu.VMEM((CNT_CH,), jnp.float32),
        pltpu.VMEM_SHARED((NT, D), jnp.float32),
        pltpu.VMEM_SHARED((NT,), jnp.float32),
        pltpu.SemaphoreType.DMA((2,)),
        pltpu.SemaphoreType.DMA((2,)),
        pltpu.SemaphoreType.DMA((2,)),
    ],
)(_sc_body)


B = 2000


def _tc_body(agg_ref, cnt_ref, x_ref, rr_ref, wl_ref, bl_ref,
             wr_ref, ws_ref, bs_ref, alpha_ref, out_ref):
    agg = agg_ref[0] + agg_ref[1]
    cnt = cnt_ref[pl.ds(0, N)] + cnt_ref[pl.ds(N, N)]
    mean = agg / jnp.maximum(cnt, 1.0)[:, None]
    h = (jnp.dot(mean, wl_ref[...], preferred_element_type=jnp.float32)
         + bl_ref[...]
         + jnp.dot(x_ref[...], wr_ref[...], preferred_element_type=jnp.float32))
    h = jnp.maximum(h, 0.0) + x_ref[...]
    score = jnp.sum(h * ws_ref[...], axis=1) + bs_ref[0, 0]
    a = jax.nn.sigmoid(alpha_ref[0, 0])
    out_ref[...] = a * rr_ref[...] + (1.0 - a) * score


def _tc_finish(agg, cnt_flat, x, rr, W_l, bl2, W_r, ws2, bs2, alpha2):
    return pl.pallas_call(
        _tc_body,
        out_shape=jax.ShapeDtypeStruct((N,), jnp.float32),
    )(agg, cnt_flat, x, rr, W_l, bl2, W_r, ws2, bs2, alpha2)


def kernel(x, edge_index, reranker_scores, W_l, b_l, W_r, W_score, b_score,
           alpha):
    agg_parts, cnt_parts = _sc_aggregate(x, edge_index[0], edge_index[1])
    return _tc_finish(
        agg_parts,
        cnt_parts,
        x,
        reranker_scores,
        W_l,
        b_l.reshape(1, D),
        W_r,
        W_score.reshape(1, D),
        b_score.reshape(1, 1),
        alpha.reshape(1, 1),
    )


# speedup vs baseline: 1.7862x; 1.0772x over previous
import functools

import jax
import jax.numpy as jnp
from jax import lax
from jax.experimental import pallas as pl
from jax.experimental.pallas import tpu as pltpu
from jax.experimental.pallas import tpu_sc as plsc

N = 10000
E = 320000
D = 128

NC = 2
NS = 16
NW = NC * NS
C = 80
NCHUNK = 125
EPW = C * NCHUNK
EP = EPW * NW
PADW = EPW - E // NW
NT = N + NS
RSTRIPE = 640
ZR = 40
CNT_CH = 2000


def _sc_body(x_hbm, src_hbm, dst_hbm, agg_out, cnt_out,
             src0_v, src1_v, src2_v, dst0_v, dst1_v, dst2_v,
             rows0_v, rows1_v, rows2_v, ones_v, zbuf, zc,
             agg_sh, cnt_sh, gsem, isem):
    s = lax.axis_index("s")
    c = lax.axis_index("c")
    wid = s * NC + c

    zero16 = jnp.zeros((16,), jnp.float32)
    one16 = jnp.ones((16,), jnp.float32)

    def fill_zbuf(i, _):
        for j in range(D // 16):
            zbuf[i, pl.ds(j * 16, 16)] = zero16
        return 0
    lax.fori_loop(0, ZR, fill_zbuf, 0)

    def fill_ones(k, _):
        ones_v[pl.ds(k * 16, 16)] = one16
        return 0
    lax.fori_loop(0, C // 16, fill_ones, 0)

    def fill_zc(k, _):
        zc[pl.ds(k * 16, 16)] = zero16
        return 0
    lax.fori_loop(0, CNT_CH // 16, fill_zc, 0)

    row0 = s * RSTRIPE
    nzero = jnp.where(s == NS - 1, (N - (NS - 1) * RSTRIPE) // ZR,
                      RSTRIPE // ZR)

    def zero_stripe(r, _):
        pltpu.sync_copy(zbuf, agg_sh.at[pl.ds(row0 + r * ZR, ZR), :])
        return 0
    lax.fori_loop(0, nzero, zero_stripe, 0)

    @pl.when(s == 0)
    def _():
        for i in range(N // CNT_CH):
            pltpu.sync_copy(zc, cnt_sh.at[pl.ds(i * CNT_CH, CNT_CH)])

    plsc.subcore_barrier()

    base0 = wid * EPW
    srcs = (src0_v, src1_v, src2_v)
    dsts = (dst0_v, dst1_v, dst2_v)
    rows = (rows0_v, rows1_v, rows2_v)
    isems = (isem.at[0], isem.at[1], isem.at[2])
    gsems = (gsem.at[0], gsem.at[1], gsem.at[2])
    NB = 3

    def load_idx(i, p):
        base = base0 + i * C
        pltpu.async_copy(src_hbm.at[pl.ds(base, C)], srcs[p], isems[p])
        pltpu.async_copy(dst_hbm.at[pl.ds(base, C)], dsts[p], isems[p])

    def wait_idx(p):
        pltpu.make_async_copy(src_hbm.at[pl.ds(0, C)], srcs[p], isems[p]).wait()
        pltpu.make_async_copy(dst_hbm.at[pl.ds(0, C)], dsts[p], isems[p]).wait()

    load_idx(0, 0)
    load_idx(1, 1)
    wait_idx(0)
    pltpu.async_copy(x_hbm.at[srcs[0]], rows[0], gsems[0])
    wait_idx(1)
    pltpu.async_copy(x_hbm.at[srcs[1]], rows[1], gsems[1])
    load_idx(2, 2)

    def outer(k, _):
        for p in range(NB):
            i = NB * k + p

            @pl.when(i < NCHUNK)
            def _():
                pltpu.make_async_copy(x_hbm.at[srcs[p]], rows[p],
                                      gsems[p]).wait()
                p2 = (p + 2) % NB

                @pl.when(i + 2 < NCHUNK)
                def _():
                    wait_idx(p2)
                    pltpu.async_copy(x_hbm.at[srcs[p2]], rows[p2],
                                     gsems[p2])

                pltpu.sync_copy(rows[p], agg_sh.at[dsts[p]], add=True)
                pltpu.sync_copy(ones_v, cnt_sh.at[dsts[p]], add=True)

                @pl.when(i + 3 < NCHUNK)
                def _():
                    load_idx(i + 3, p)
        return 0
    lax.fori_loop(0, (NCHUNK + NB - 1) // NB, outer, 0)

    plsc.subcore_barrier()

    nout = jnp.where(s == NS - 1, (N - (NS - 1) * RSTRIPE) // ZR,
                     RSTRIPE // ZR)

    def out_stripe(r, _):
        rb = row0 + r * ZR
        pltpu.sync_copy(agg_sh.at[pl.ds(rb, ZR), :], zbuf)
        pltpu.sync_copy(zbuf, agg_out.at[c, pl.ds(rb, ZR), :])
        return 0
    lax.fori_loop(0, nout, out_stripe, 0)

    @pl.when(s == 0)
    def _():
        for i in range(N // CNT_CH):
            pltpu.sync_copy(cnt_sh.at[pl.ds(i * CNT_CH, CNT_CH)], zc)
            pltpu.sync_copy(zc, cnt_out.at[pl.ds(c * N + i * CNT_CH, CNT_CH)])


_sc_aggregate = functools.partial(
    pl.kernel,
    out_type=[
        jax.ShapeDtypeStruct((NC, N, D), jnp.float32),
        jax.ShapeDtypeStruct((NC * N,), jnp.float32),
    ],
    mesh=plsc.VectorSubcoreMesh(core_axis_name="c", subcore_axis_name="s"),
    scratch_types=[
        pltpu.VMEM((C,), jnp.int32),
        pltpu.VMEM((C,), jnp.int32),
        pltpu.VMEM((C,), jnp.int32),
        pltpu.VMEM((C,), jnp.int32),
        pltpu.VMEM((C,), jnp.int32),
        pltpu.VMEM((C,), jnp.int32),
        pltpu.VMEM((C, D), jnp.float32),
        pltpu.VMEM((C, D), jnp.float32),
        pltpu.VMEM((C, D), jnp.float32),
        pltpu.VMEM((C,), jnp.float32),
        pltpu.VMEM((ZR, D), jnp.float32),
        pltpu.VMEM((CNT_CH,), jnp.float32),
        pltpu.VMEM_SHARED((NT, D), jnp.float32),
        pltpu.VMEM_SHARED((NT,), jnp.float32),
        pltpu.SemaphoreType.DMA((3,)),
        pltpu.SemaphoreType.DMA((3,)),
    ],
)(_sc_body)


B = 2000


def _tc_body(agg_ref, cnt_ref, x_ref, rr_ref, wl_ref, bl_ref,
             wr_ref, ws_ref, bs_ref, alpha_ref, out_ref):
    agg = agg_ref[0] + agg_ref[1]
    cnt = cnt_ref[pl.ds(0, N)] + cnt_ref[pl.ds(N, N)]
    mean = agg / jnp.maximum(cnt, 1.0)[:, None]
    h = (jnp.dot(mean, wl_ref[...], preferred_element_type=jnp.float32)
         + bl_ref[...]
         + jnp.dot(x_ref[...], wr_ref[...], preferred_element_type=jnp.float32))
    h = jnp.maximum(h, 0.0) + x_ref[...]
    score = jnp.sum(h * ws_ref[...], axis=1) + bs_ref[0, 0]
    a = jax.nn.sigmoid(alpha_ref[0, 0])
    out_ref[...] = a * rr_ref[...] + (1.0 - a) * score


def _tc_finish(agg, cnt_flat, x, rr, W_l, bl2, W_r, ws2, bs2, alpha2):
    return pl.pallas_call(
        _tc_body,
        out_shape=jax.ShapeDtypeStruct((N,), jnp.float32),
    )(agg, cnt_flat, x, rr, W_l, bl2, W_r, ws2, bs2, alpha2)


def kernel(x, edge_index, reranker_scores, W_l, b_l, W_r, W_score, b_score,
           alpha):
    agg_parts, cnt_parts = _sc_aggregate(x, edge_index[0], edge_index[1])
    return _tc_finish(
        agg_parts,
        cnt_parts,
        x,
        reranker_scores,
        W_l,
        b_l.reshape(1, D),
        W_r,
        W_score.reshape(1, D),
        b_score.reshape(1, 1),
        alpha.reshape(1, 1),
    )


# speedup vs baseline: 1.7900x; 1.0021x over previous
import functools

import jax
import jax.numpy as jnp
from jax import lax
from jax.experimental import pallas as pl
from jax.experimental.pallas import tpu as pltpu
from jax.experimental.pallas import tpu_sc as plsc

N = 10000
E = 320000
D = 128

NC = 2
NS = 16
NW = NC * NS
C = 80
NCHUNK = 125
EPW = C * NCHUNK
EP = EPW * NW
PADW = EPW - E // NW
NT = N + NS
RSTRIPE = 640
ZR = 40
CNT_CH = 2000


def _sc_body(x_hbm, src_hbm, dst_hbm, agg_out, cnt_out,
             src0_v, src1_v, src2_v, src3_v, dst0_v, dst1_v, dst2_v, dst3_v,
             rows0_v, rows1_v, rows2_v, rows3_v, ones_v, zbuf, zc,
             agg_sh, cnt_sh, gsem, isem):
    s = lax.axis_index("s")
    c = lax.axis_index("c")
    wid = s * NC + c

    zero16 = jnp.zeros((16,), jnp.float32)
    one16 = jnp.ones((16,), jnp.float32)

    def fill_zbuf(i, _):
        for j in range(D // 16):
            zbuf[i, pl.ds(j * 16, 16)] = zero16
        return 0
    lax.fori_loop(0, ZR, fill_zbuf, 0)

    def fill_ones(k, _):
        ones_v[pl.ds(k * 16, 16)] = one16
        return 0
    lax.fori_loop(0, C // 16, fill_ones, 0)

    def fill_zc(k, _):
        zc[pl.ds(k * 16, 16)] = zero16
        return 0
    lax.fori_loop(0, CNT_CH // 16, fill_zc, 0)

    row0 = s * RSTRIPE
    nzero = jnp.where(s == NS - 1, (N - (NS - 1) * RSTRIPE) // ZR,
                      RSTRIPE // ZR)

    def zero_stripe(r, _):
        pltpu.sync_copy(zbuf, agg_sh.at[pl.ds(row0 + r * ZR, ZR), :])
        return 0
    lax.fori_loop(0, nzero, zero_stripe, 0)

    @pl.when(s == 0)
    def _():
        for i in range(N // CNT_CH):
            pltpu.sync_copy(zc, cnt_sh.at[pl.ds(i * CNT_CH, CNT_CH)])

    plsc.subcore_barrier()

    base0 = wid * EPW
    srcs = (src0_v, src1_v, src2_v, src3_v)
    dsts = (dst0_v, dst1_v, dst2_v, dst3_v)
    rows = (rows0_v, rows1_v, rows2_v, rows3_v)
    isems = (isem.at[0], isem.at[1], isem.at[2], isem.at[3])
    gsems = (gsem.at[0], gsem.at[1], gsem.at[2], gsem.at[3])
    NB = 4

    def load_idx(i, p):
        base = base0 + i * C
        pltpu.async_copy(src_hbm.at[pl.ds(base, C)], srcs[p], isems[p])
        pltpu.async_copy(dst_hbm.at[pl.ds(base, C)], dsts[p], isems[p])

    def wait_idx(p):
        pltpu.make_async_copy(src_hbm.at[pl.ds(0, C)], srcs[p], isems[p]).wait()
        pltpu.make_async_copy(dst_hbm.at[pl.ds(0, C)], dsts[p], isems[p]).wait()

    load_idx(0, 0)
    load_idx(1, 1)
    load_idx(2, 2)
    wait_idx(0)
    pltpu.async_copy(x_hbm.at[srcs[0]], rows[0], gsems[0])
    wait_idx(1)
    pltpu.async_copy(x_hbm.at[srcs[1]], rows[1], gsems[1])
    wait_idx(2)
    pltpu.async_copy(x_hbm.at[srcs[2]], rows[2], gsems[2])
    load_idx(3, 3)

    def outer(k, _):
        for p in range(NB):
            i = NB * k + p

            @pl.when(i < NCHUNK)
            def _():
                pltpu.make_async_copy(x_hbm.at[srcs[p]], rows[p],
                                      gsems[p]).wait()
                p2 = (p + 3) % NB

                @pl.when(i + 3 < NCHUNK)
                def _():
                    wait_idx(p2)
                    pltpu.async_copy(x_hbm.at[srcs[p2]], rows[p2],
                                     gsems[p2])

                pltpu.sync_copy(rows[p], agg_sh.at[dsts[p]], add=True)
                pltpu.sync_copy(ones_v, cnt_sh.at[dsts[p]], add=True)

                @pl.when(i + 4 < NCHUNK)
                def _():
                    load_idx(i + 4, p)
        return 0
    lax.fori_loop(0, (NCHUNK + NB - 1) // NB, outer, 0)

    plsc.subcore_barrier()

    nout = jnp.where(s == NS - 1, (N - (NS - 1) * RSTRIPE) // ZR,
                     RSTRIPE // ZR)

    def out_stripe(r, _):
        rb = row0 + r * ZR
        pltpu.sync_copy(agg_sh.at[pl.ds(rb, ZR), :], zbuf)
        pltpu.sync_copy(zbuf, agg_out.at[c, pl.ds(rb, ZR), :])
        return 0
    lax.fori_loop(0, nout, out_stripe, 0)

    @pl.when(s == 0)
    def _():
        for i in range(N // CNT_CH):
            pltpu.sync_copy(cnt_sh.at[pl.ds(i * CNT_CH, CNT_CH)], zc)
            pltpu.sync_copy(zc, cnt_out.at[pl.ds(c * N + i * CNT_CH, CNT_CH)])


_sc_aggregate = functools.partial(
    pl.kernel,
    out_type=[
        jax.ShapeDtypeStruct((NC, N, D), jnp.float32),
        jax.ShapeDtypeStruct((NC * N,), jnp.float32),
    ],
    mesh=plsc.VectorSubcoreMesh(core_axis_name="c", subcore_axis_name="s"),
    scratch_types=[
        pltpu.VMEM((C,), jnp.int32),
        pltpu.VMEM((C,), jnp.int32),
        pltpu.VMEM((C,), jnp.int32),
        pltpu.VMEM((C,), jnp.int32),
        pltpu.VMEM((C,), jnp.int32),
        pltpu.VMEM((C,), jnp.int32),
        pltpu.VMEM((C,), jnp.int32),
        pltpu.VMEM((C,), jnp.int32),
        pltpu.VMEM((C, D), jnp.float32),
        pltpu.VMEM((C, D), jnp.float32),
        pltpu.VMEM((C, D), jnp.float32),
        pltpu.VMEM((C, D), jnp.float32),
        pltpu.VMEM((C,), jnp.float32),
        pltpu.VMEM((ZR, D), jnp.float32),
        pltpu.VMEM((CNT_CH,), jnp.float32),
        pltpu.VMEM_SHARED((NT, D), jnp.float32),
        pltpu.VMEM_SHARED((NT,), jnp.float32),
        pltpu.SemaphoreType.DMA((4,)),
        pltpu.SemaphoreType.DMA((4,)),
    ],
)(_sc_body)


B = 2000


def _tc_body(agg_ref, cnt_ref, x_ref, rr_ref, wl_ref, bl_ref,
             wr_ref, ws_ref, bs_ref, alpha_ref, out_ref):
    agg = agg_ref[0] + agg_ref[1]
    cnt = cnt_ref[pl.ds(0, N)] + cnt_ref[pl.ds(N, N)]
    mean = agg / jnp.maximum(cnt, 1.0)[:, None]
    h = (jnp.dot(mean, wl_ref[...], preferred_element_type=jnp.float32)
         + bl_ref[...]
         + jnp.dot(x_ref[...], wr_ref[...], preferred_element_type=jnp.float32))
    h = jnp.maximum(h, 0.0) + x_ref[...]
    score = jnp.sum(h * ws_ref[...], axis=1) + bs_ref[0, 0]
    a = jax.nn.sigmoid(alpha_ref[0, 0])
    out_ref[...] = a * rr_ref[...] + (1.0 - a) * score


def _tc_finish(agg, cnt_flat, x, rr, W_l, bl2, W_r, ws2, bs2, alpha2):
    return pl.pallas_call(
        _tc_body,
        out_shape=jax.ShapeDtypeStruct((N,), jnp.float32),
    )(agg, cnt_flat, x, rr, W_l, bl2, W_r, ws2, bs2, alpha2)


def kernel(x, edge_index, reranker_scores, W_l, b_l, W_r, W_score, b_score,
           alpha):
    agg_parts, cnt_parts = _sc_aggregate(x, edge_index[0], edge_index[1])
    return _tc_finish(
        agg_parts,
        cnt_parts,
        x,
        reranker_scores,
        W_l,
        b_l.reshape(1, D),
        W_r,
        W_score.reshape(1, D),
        b_score.reshape(1, 1),
        alpha.reshape(1, 1),
    )


# speedup vs baseline: 1.9063x; 1.0650x over previous
import functools

import jax
import jax.numpy as jnp
from jax import lax
from jax.experimental import pallas as pl
from jax.experimental.pallas import tpu as pltpu
from jax.experimental.pallas import tpu_sc as plsc

N = 10000
E = 320000
D = 128

NC = 2
NS = 16
NW = NC * NS
C = 80
NCHUNK = 125
EPW = C * NCHUNK
EP = EPW * NW
PADW = EPW - E // NW
NT = N + NS
RSTRIPE = 640
ZR = 40
CNT_CH = 2000


def _sc_body(x_hbm, src_hbm, dst_hbm, agg_out, cnt_out,
             src0_v, src1_v, src2_v, src3_v, dst0_v, dst1_v, dst2_v, dst3_v,
             rows0_v, rows1_v, rows2_v, rows3_v, ones_v, zbuf, zc,
             agg_sh, cnt_sh, gsem, isem):
    s = lax.axis_index("s")
    c = lax.axis_index("c")
    wid = s * NC + c

    zero16 = jnp.zeros((16,), jnp.float32)
    one16 = jnp.ones((16,), jnp.float32)

    def fill_zbuf(i, _):
        for j in range(D // 16):
            zbuf[i, pl.ds(j * 16, 16)] = zero16
        return 0
    lax.fori_loop(0, ZR, fill_zbuf, 0)

    def fill_ones(k, _):
        ones_v[pl.ds(k * 16, 16)] = one16
        return 0
    lax.fori_loop(0, C // 16, fill_ones, 0)

    def fill_zc(k, _):
        zc[pl.ds(k * 16, 16)] = zero16
        return 0
    lax.fori_loop(0, CNT_CH // 16, fill_zc, 0)

    row0 = s * RSTRIPE
    nzero = jnp.where(s == NS - 1, (N - (NS - 1) * RSTRIPE) // ZR,
                      RSTRIPE // ZR)

    def zero_stripe(r, _):
        pltpu.sync_copy(zbuf, agg_sh.at[pl.ds(row0 + r * ZR, ZR), :])
        return 0
    lax.fori_loop(0, nzero, zero_stripe, 0)

    @pl.when(s == 0)
    def _():
        for i in range(N // CNT_CH):
            pltpu.sync_copy(zc, cnt_sh.at[pl.ds(i * CNT_CH, CNT_CH)])

    plsc.subcore_barrier()

    base0 = wid * EPW
    srcs = (src0_v, src1_v, src2_v, src3_v)
    dsts = (dst0_v, dst1_v, dst2_v, dst3_v)
    rows = (rows0_v, rows1_v, rows2_v, rows3_v)
    isems = (isem.at[0], isem.at[1], isem.at[2], isem.at[3])
    gsems = (gsem.at[0], gsem.at[1], gsem.at[2], gsem.at[3])
    NB = 4

    def load_idx(i, p):
        base = base0 + i * C
        pltpu.async_copy(src_hbm.at[pl.ds(base, C)], srcs[p], isems[p])
        pltpu.async_copy(dst_hbm.at[pl.ds(base, C)], dsts[p], isems[p])

    def wait_idx(p):
        pltpu.make_async_copy(src_hbm.at[pl.ds(0, C)], srcs[p], isems[p]).wait()
        pltpu.make_async_copy(dst_hbm.at[pl.ds(0, C)], dsts[p], isems[p]).wait()

    load_idx(0, 0)
    load_idx(1, 1)
    load_idx(2, 2)
    wait_idx(0)
    pltpu.async_copy(x_hbm.at[srcs[0]], rows[0], gsems[0])
    wait_idx(1)
    pltpu.async_copy(x_hbm.at[srcs[1]], rows[1], gsems[1])
    wait_idx(2)
    pltpu.async_copy(x_hbm.at[srcs[2]], rows[2], gsems[2])
    load_idx(3, 3)

    def outer(k, _):
        for p in range(NB):
            i = NB * k + p

            @pl.when(i < NCHUNK)
            def _():
                pltpu.make_async_copy(x_hbm.at[srcs[p]], rows[p],
                                      gsems[p]).wait()
                p2 = (p + 3) % NB

                @pl.when(i + 3 < NCHUNK)
                def _():
                    wait_idx(p2)
                    pltpu.async_copy(x_hbm.at[srcs[p2]], rows[p2],
                                     gsems[p2])

                pltpu.sync_copy(rows[p], agg_sh.at[dsts[p]], add=True)
                pltpu.sync_copy(ones_v, cnt_sh.at[dsts[p]], add=True)

                @pl.when(i + 4 < NCHUNK)
                def _():
                    load_idx(i + 4, p)
        return 0
    lax.fori_loop(0, (NCHUNK + NB - 1) // NB, outer, 0)

    plsc.subcore_barrier()

    nout = jnp.where(s == NS - 1, (N - (NS - 1) * RSTRIPE) // ZR,
                     RSTRIPE // ZR)

    def out_stripe(r, _):
        rb = row0 + r * ZR
        pltpu.sync_copy(agg_sh.at[pl.ds(rb, ZR), :], zbuf)
        pltpu.sync_copy(zbuf, agg_out.at[c, pl.ds(rb, ZR), :])
        return 0
    lax.fori_loop(0, nout, out_stripe, 0)

    @pl.when(s == 0)
    def _():
        for i in range(N // CNT_CH):
            pltpu.sync_copy(cnt_sh.at[pl.ds(i * CNT_CH, CNT_CH)], zc)
            pltpu.sync_copy(zc, cnt_out.at[pl.ds(c * N + i * CNT_CH, CNT_CH)])


_sc_aggregate = functools.partial(
    pl.kernel,
    out_type=[
        jax.ShapeDtypeStruct((NC, N, D), jnp.float32),
        jax.ShapeDtypeStruct((NC * N,), jnp.float32),
    ],
    mesh=plsc.VectorSubcoreMesh(core_axis_name="c", subcore_axis_name="s"),
    scratch_types=[
        pltpu.VMEM((C,), jnp.int32),
        pltpu.VMEM((C,), jnp.int32),
        pltpu.VMEM((C,), jnp.int32),
        pltpu.VMEM((C,), jnp.int32),
        pltpu.VMEM((C,), jnp.int32),
        pltpu.VMEM((C,), jnp.int32),
        pltpu.VMEM((C,), jnp.int32),
        pltpu.VMEM((C,), jnp.int32),
        pltpu.VMEM((C, D), jnp.float32),
        pltpu.VMEM((C, D), jnp.float32),
        pltpu.VMEM((C, D), jnp.float32),
        pltpu.VMEM((C, D), jnp.float32),
        pltpu.VMEM((C,), jnp.float32),
        pltpu.VMEM((ZR, D), jnp.float32),
        pltpu.VMEM((CNT_CH,), jnp.float32),
        pltpu.VMEM_SHARED((NT, D), jnp.float32),
        pltpu.VMEM_SHARED((NT,), jnp.float32),
        pltpu.SemaphoreType.DMA((4,)),
        pltpu.SemaphoreType.DMA((4,)),
    ],
)(_sc_body)


def _split_body(ei_ref, src_ref, dst_ref):
    ei = ei_ref[...]
    src_ref[...] = ei[0]
    dst_ref[...] = ei[1]


def _split_edges(edge_index):
    return pl.pallas_call(
        _split_body,
        out_shape=[
            jax.ShapeDtypeStruct((E,), jnp.int32),
            jax.ShapeDtypeStruct((E,), jnp.int32),
        ],
    )(edge_index)


B = 2000


def _tc_body(agg_ref, cnt_ref, x_ref, rr_ref, wl_ref, bl_ref,
             wr_ref, ws_ref, bs_ref, alpha_ref, out_ref):
    agg = agg_ref[0] + agg_ref[1]
    cnt = cnt_ref[pl.ds(0, N)] + cnt_ref[pl.ds(N, N)]
    mean = agg / jnp.maximum(cnt, 1.0)[:, None]
    h = (jnp.dot(mean, wl_ref[...], preferred_element_type=jnp.float32)
         + bl_ref[...]
         + jnp.dot(x_ref[...], wr_ref[...], preferred_element_type=jnp.float32))
    h = jnp.maximum(h, 0.0) + x_ref[...]
    score = jnp.sum(h * ws_ref[...], axis=1) + bs_ref[0, 0]
    a = jax.nn.sigmoid(alpha_ref[0, 0])
    out_ref[...] = a * rr_ref[...] + (1.0 - a) * score


def _tc_finish(agg, cnt_flat, x, rr, W_l, bl2, W_r, ws2, bs2, alpha2):
    return pl.pallas_call(
        _tc_body,
        out_shape=jax.ShapeDtypeStruct((N,), jnp.float32),
    )(agg, cnt_flat, x, rr, W_l, bl2, W_r, ws2, bs2, alpha2)


def kernel(x, edge_index, reranker_scores, W_l, b_l, W_r, W_score, b_score,
           alpha):
    srcp, dstp = _split_edges(edge_index)
    agg_parts, cnt_parts = _sc_aggregate(x, srcp, dstp)
    return _tc_finish(
        agg_parts,
        cnt_parts,
        x,
        reranker_scores,
        W_l,
        b_l.reshape(1, D),
        W_r,
        W_score.reshape(1, D),
        b_score.reshape(1, 1),
        alpha.reshape(1, 1),
    )


# speedup vs baseline: 1.9268x; 1.0107x over previous
import functools

import jax
import jax.numpy as jnp
from jax import lax
from jax.experimental import pallas as pl
from jax.experimental.pallas import tpu as pltpu
from jax.experimental.pallas import tpu_sc as plsc

N = 10000
E = 320000
D = 128

NC = 2
NS = 16
NW = NC * NS
C = 80
NCHUNK = 125
EPW = C * NCHUNK
EP = EPW * NW
PADW = EPW - E // NW
NT = N + NS
RSTRIPE = 640
ZR = 80
CNT_CH = 2000


def _sc_body(x_hbm, src_hbm, dst_hbm, agg_out, cnt_out,
             src0_v, src1_v, src2_v, dst0_v, dst1_v, dst2_v,
             rows0_v, rows1_v, rows2_v, ones_v, zbuf, zc,
             agg_sh, cnt_sh, gsem, isem):
    s = lax.axis_index("s")
    c = lax.axis_index("c")
    wid = s * NC + c

    zero16 = jnp.zeros((16,), jnp.float32)
    one16 = jnp.ones((16,), jnp.float32)

    def fill_zbuf(i, _):
        for j in range(D // 16):
            zbuf[i, pl.ds(j * 16, 16)] = zero16
        return 0
    lax.fori_loop(0, ZR, fill_zbuf, 0)

    def fill_ones(k, _):
        ones_v[pl.ds(k * 16, 16)] = one16
        return 0
    lax.fori_loop(0, C // 16, fill_ones, 0)

    def fill_zc(k, _):
        zc[pl.ds(k * 16, 16)] = zero16
        return 0
    lax.fori_loop(0, CNT_CH // 16, fill_zc, 0)

    row0 = s * RSTRIPE
    nzero = jnp.where(s == NS - 1, (N - (NS - 1) * RSTRIPE) // ZR,
                      RSTRIPE // ZR)

    def zero_stripe(r, _):
        pltpu.sync_copy(zbuf, agg_sh.at[pl.ds(row0 + r * ZR, ZR), :])
        return 0
    lax.fori_loop(0, nzero, zero_stripe, 0)

    @pl.when(s == 0)
    def _():
        for i in range(N // CNT_CH):
            pltpu.sync_copy(zc, cnt_sh.at[pl.ds(i * CNT_CH, CNT_CH)])

    plsc.subcore_barrier()

    base0 = wid * EPW
    srcs = (src0_v, src1_v, src2_v)
    dsts = (dst0_v, dst1_v, dst2_v)
    rows = (rows0_v, rows1_v, rows2_v)
    isems = (isem.at[0], isem.at[1], isem.at[2])
    gsems = (gsem.at[0], gsem.at[1], gsem.at[2])
    NB = 3

    def load_idx(i, p):
        base = base0 + i * C
        pltpu.async_copy(src_hbm.at[pl.ds(base, C)], srcs[p], isems[p])
        pltpu.async_copy(dst_hbm.at[pl.ds(base, C)], dsts[p], isems[p])

    def wait_idx(p):
        pltpu.make_async_copy(src_hbm.at[pl.ds(0, C)], srcs[p], isems[p]).wait()
        pltpu.make_async_copy(dst_hbm.at[pl.ds(0, C)], dsts[p], isems[p]).wait()

    load_idx(0, 0)
    load_idx(1, 1)
    wait_idx(0)
    pltpu.async_copy(x_hbm.at[srcs[0]], rows[0], gsems[0])
    wait_idx(1)
    pltpu.async_copy(x_hbm.at[srcs[1]], rows[1], gsems[1])
    load_idx(2, 2)

    def outer(k, _):
        for p in range(NB):
            i = NB * k + p

            @pl.when(i < NCHUNK)
            def _():
                pltpu.make_async_copy(x_hbm.at[srcs[p]], rows[p],
                                      gsems[p]).wait()
                p2 = (p + 2) % NB

                @pl.when(i + 2 < NCHUNK)
                def _():
                    wait_idx(p2)
                    pltpu.async_copy(x_hbm.at[srcs[p2]], rows[p2],
                                     gsems[p2])

                pltpu.sync_copy(rows[p], agg_sh.at[dsts[p]], add=True)
                pltpu.sync_copy(ones_v, cnt_sh.at[dsts[p]], add=True)

                @pl.when(i + 3 < NCHUNK)
                def _():
                    load_idx(i + 3, p)
        return 0
    lax.fori_loop(0, (NCHUNK + NB - 1) // NB, outer, 0)

    plsc.subcore_barrier()

    nout = jnp.where(s == NS - 1, (N - (NS - 1) * RSTRIPE) // ZR,
                     RSTRIPE // ZR)

    def out_stripe(r, _):
        rb = row0 + r * ZR
        pltpu.sync_copy(agg_sh.at[pl.ds(rb, ZR), :], zbuf)
        pltpu.sync_copy(zbuf, agg_out.at[c, pl.ds(rb, ZR), :])
        return 0
    lax.fori_loop(0, nout, out_stripe, 0)

    @pl.when(s == 0)
    def _():
        for i in range(N // CNT_CH):
            pltpu.sync_copy(cnt_sh.at[pl.ds(i * CNT_CH, CNT_CH)], zc)
            pltpu.sync_copy(zc, cnt_out.at[pl.ds(c * N + i * CNT_CH, CNT_CH)])


_sc_aggregate = functools.partial(
    pl.kernel,
    out_type=[
        jax.ShapeDtypeStruct((NC, N, D), jnp.float32),
        jax.ShapeDtypeStruct((NC * N,), jnp.float32),
    ],
    mesh=plsc.VectorSubcoreMesh(core_axis_name="c", subcore_axis_name="s"),
    scratch_types=[
        pltpu.VMEM((C,), jnp.int32),
        pltpu.VMEM((C,), jnp.int32),
        pltpu.VMEM((C,), jnp.int32),
        pltpu.VMEM((C,), jnp.int32),
        pltpu.VMEM((C,), jnp.int32),
        pltpu.VMEM((C,), jnp.int32),
        pltpu.VMEM((C, D), jnp.float32),
        pltpu.VMEM((C, D), jnp.float32),
        pltpu.VMEM((C, D), jnp.float32),
        pltpu.VMEM((C,), jnp.float32),
        pltpu.VMEM((ZR, D), jnp.float32),
        pltpu.VMEM((CNT_CH,), jnp.float32),
        pltpu.VMEM_SHARED((NT, D), jnp.float32),
        pltpu.VMEM_SHARED((NT,), jnp.float32),
        pltpu.SemaphoreType.DMA((3,)),
        pltpu.SemaphoreType.DMA((3,)),
    ],
)(_sc_body)


def _split_body(ei_ref, src_ref, dst_ref):
    ei = ei_ref[...]
    src_ref[...] = ei[0]
    dst_ref[...] = ei[1]


def _split_edges(edge_index):
    return pl.pallas_call(
        _split_body,
        out_shape=[
            jax.ShapeDtypeStruct((E,), jnp.int32),
            jax.ShapeDtypeStruct((E,), jnp.int32),
        ],
    )(edge_index)


B = 2000


def _tc_body(agg_ref, cnt_ref, x_ref, rr_ref, wl_ref, bl_ref,
             wr_ref, ws_ref, bs_ref, alpha_ref, out_ref):
    agg = agg_ref[0] + agg_ref[1]
    cnt = cnt_ref[pl.ds(0, N)] + cnt_ref[pl.ds(N, N)]
    mean = agg / jnp.maximum(cnt, 1.0)[:, None]
    h = (jnp.dot(mean, wl_ref[...], preferred_element_type=jnp.float32)
         + bl_ref[...]
         + jnp.dot(x_ref[...], wr_ref[...], preferred_element_type=jnp.float32))
    h = jnp.maximum(h, 0.0) + x_ref[...]
    score = jnp.sum(h * ws_ref[...], axis=1) + bs_ref[0, 0]
    a = jax.nn.sigmoid(alpha_ref[0, 0])
    out_ref[...] = a * rr_ref[...] + (1.0 - a) * score


def _tc_finish(agg, cnt_flat, x, rr, W_l, bl2, W_r, ws2, bs2, alpha2):
    return pl.pallas_call(
        _tc_body,
        out_shape=jax.ShapeDtypeStruct((N,), jnp.float32),
    )(agg, cnt_flat, x, rr, W_l, bl2, W_r, ws2, bs2, alpha2)


def kernel(x, edge_index, reranker_scores, W_l, b_l, W_r, W_score, b_score,
           alpha):
    srcp, dstp = _split_edges(edge_index)
    agg_parts, cnt_parts = _sc_aggregate(x, srcp, dstp)
    return _tc_finish(
        agg_parts,
        cnt_parts,
        x,
        reranker_scores,
        W_l,
        b_l.reshape(1, D),
        W_r,
        W_score.reshape(1, D),
        b_score.reshape(1, 1),
        alpha.reshape(1, 1),
    )
